# baseline (speedup 1.0000x reference)
import functools

import jax
import jax.numpy as jnp
from jax import lax
from jax.experimental import pallas as pl
from jax.experimental.pallas import tpu as pltpu
from jax.experimental.pallas import tpu_sc as plsc

LANES = 16
SUB = 8
LN = 128
TILE = SUB * LN


def _pos_embed_body(nb, h, w, d, f, nc,
                    colf_hbm, rowf_hbm, depf_hbm, out_hbm,
                    colv, rowv, depv, slab0, slab1, slab2, slab3,
                    sem0, sem1, sem2, sem3, sems):
    dt_n = d // SUB
    ct_n = (3 * f) // LN
    slab_len = dt_n * ct_n * TILE
    jn = f // LANES

    hh = lax.axis_index("s") * nc + lax.axis_index("c")

    cp0 = pltpu.async_copy(colf_hbm.at[pl.ds(0, w * f)], colv, sems)
    cp1 = pltpu.async_copy(depf_hbm.at[pl.ds(0, d * f)], depv, sems)
    cp2 = pltpu.async_copy(rowf_hbm.at[pl.ds(hh * f, f)], rowv, sems)
    cp0.wait()
    cp1.wait()
    cp2.wait()

    slabs = (slab0, slab1, slab2, slab3)
    semv = (sem0, sem1, sem2, sem3)
    nbuf = len(slabs)

    row_regs = [rowv[pl.ds(j * LANES, LANES)] for j in range(jn)]
    for buf in slabs:
        for dt in range(dt_n):
            b1 = (dt * ct_n + 1) * TILE
            b2 = (dt * ct_n + 2) * TILE
            for dr in range(SUB):
                for j in range(jn):
                    o = dr * LN + j * LANES
                    buf[pl.ds(b1 + o, LANES)] = row_regs[j]
                    buf[pl.ds(b2 + o, LANES)] = depv[pl.ds(dt * TILE + o,
                                                           LANES)]

    def patch_col(buf, wq):
        col_regs = [colv[pl.ds(wq * f + j * LANES, LANES)] for j in range(jn)]
        for dt in range(dt_n):
            b0 = dt * ct_n * TILE
            for dr in range(SUB):
                for j in range(jn):
                    buf[pl.ds(b0 + dr * LN + j * LANES, LANES)] = col_regs[j]

    def fire(buf, wq, sem):
        for b in range(nb):
            off = ((b * h + hh) * w + wq) * slab_len
            pltpu.async_copy(buf, out_hbm.at[pl.ds(off, slab_len)], sem)

    def drain(buf, sem):
        for _ in range(nb):
            pltpu.make_async_copy(
                buf, out_hbm.at[pl.ds(0, slab_len)], sem).wait()

    for i in range(nbuf):
        patch_col(slabs[i], i)
        fire(slabs[i], i, semv[i])

    def grp(k, carry):
        wq = nbuf * k
        for i in range(nbuf):
            drain(slabs[i], semv[i])
            patch_col(slabs[i], wq + i)
            fire(slabs[i], wq + i, semv[i])
        return carry

    lax.fori_loop(1, w // nbuf, grp, 0)

    for i in range(nbuf):
        drain(slabs[i], semv[i])


def kernel(tensor_list, row_embed_weight, col_embed_weight, depth_embed_weight):
    x = tensor_list
    h, w, d = x.shape[-3], x.shape[-2], x.shape[-1]
    nb = x.shape[0]
    f = row_embed_weight.shape[-1]
    n_chan = 3 * f

    info = plsc.get_sparse_core_info()
    nc, ns = info.num_cores, info.num_subcores
    assert nc * ns == h, "one vector subcore per h plane"

    colf = col_embed_weight.reshape(-1)
    rowf = row_embed_weight.reshape(-1)
    depf = depth_embed_weight.reshape(-1)

    dt_n = d // SUB
    ct_n = n_chan // LN
    total = nb * h * w * dt_n * ct_n * TILE

    run = pl.kernel(
        functools.partial(_pos_embed_body, nb, h, w, d, f, nc),
        mesh=plsc.VectorSubcoreMesh(core_axis_name="c", subcore_axis_name="s"),
        out_type=jax.ShapeDtypeStruct((total,), jnp.float32),
        scratch_types=[
            pltpu.VMEM((w * f,), jnp.float32),
            pltpu.VMEM((f,), jnp.float32),
            pltpu.VMEM((d * f,), jnp.float32),
            pltpu.VMEM((dt_n * ct_n * TILE,), jnp.float32),
            pltpu.VMEM((dt_n * ct_n * TILE,), jnp.float32),
            pltpu.VMEM((dt_n * ct_n * TILE,), jnp.float32),
            pltpu.VMEM((dt_n * ct_n * TILE,), jnp.float32),
            pltpu.SemaphoreType.DMA,
            pltpu.SemaphoreType.DMA,
            pltpu.SemaphoreType.DMA,
            pltpu.SemaphoreType.DMA,
            pltpu.SemaphoreType.DMA,
        ],
    )
    out = run(colf, rowf, depf)
    out7 = out.reshape(nb, h, w, dt_n, ct_n, SUB, LN)
    out5 = out7.transpose(0, 4, 6, 1, 2, 3, 5).reshape(nb, n_chan, h, w, d)
    return out5


# speedup vs baseline: 1.0301x; 1.0301x over previous
import functools

import jax
import jax.numpy as jnp
from jax import lax
from jax.experimental import pallas as pl
from jax.experimental.pallas import tpu as pltpu
from jax.experimental.pallas import tpu_sc as plsc

LANES = 16
SUB = 8
LN = 128
TILE = SUB * LN


def _pos_embed_body(nb, h, w, d, f, nc,
                    colf_hbm, rowf_hbm, depf_hbm, out_hbm,
                    colv, rowv, depv, rowt, colt0, colt1,
                    semc0, semc1, semrd):
    dt_n = d // SUB
    ct_n = (3 * f) // LN
    slab_len = dt_n * ct_n * TILE
    jn = f // LANES

    hh = lax.axis_index("s") * nc + lax.axis_index("c")

    cp0 = pltpu.async_copy(colf_hbm.at[pl.ds(0, w * f)], colv, semrd)
    cp1 = pltpu.async_copy(depf_hbm.at[pl.ds(0, d * f)], depv, semrd)
    cp2 = pltpu.async_copy(rowf_hbm.at[pl.ds(hh * f, f)], rowv, semrd)
    cp0.wait()
    cp1.wait()
    cp2.wait()

    row_regs = [rowv[pl.ds(j * LANES, LANES)] for j in range(jn)]
    for dr in range(SUB):
        for j in range(jn):
            rowt[pl.ds(dr * LN + j * LANES, LANES)] = row_regs[j]

    def build_col(buf, wq):
        col_regs = [colv[pl.ds(wq * f + j * LANES, LANES)] for j in range(jn)]
        for dr in range(SUB):
            for j in range(jn):
                buf[pl.ds(dr * LN + j * LANES, LANES)] = col_regs[j]

    def fire(colt, wq, semc):
        for b in range(nb):
            base = ((b * h + hh) * w + wq) * slab_len
            for dt in range(dt_n):
                off = base + dt * ct_n * TILE
                pltpu.async_copy(colt, out_hbm.at[pl.ds(off, TILE)], semc)
                pltpu.async_copy(rowt, out_hbm.at[pl.ds(off + TILE, TILE)],
                                 semrd)
                pltpu.async_copy(depv.at[pl.ds(dt * TILE, TILE)],
                                 out_hbm.at[pl.ds(off + 2 * TILE, TILE)],
                                 semrd)

    def drain(buf, sem, n):
        for _ in range(n):
            pltpu.make_async_copy(buf, out_hbm.at[pl.ds(0, TILE)], sem).wait()

    col_fires = nb * dt_n

    build_col(colt0, 0)
    fire(colt0, 0, semc0)
    build_col(colt1, 1)
    fire(colt1, 1, semc1)

    def pair(k, carry):
        wq = 2 * k
        drain(colt0, semc0, col_fires)
        build_col(colt0, wq)
        fire(colt0, wq, semc0)
        drain(colt1, semc1, col_fires)
        build_col(colt1, wq + 1)
        fire(colt1, wq + 1, semc1)
        return carry

    lax.fori_loop(1, w // 2, pair, 0)

    drain(colt0, semc0, col_fires)
    drain(colt1, semc1, col_fires)

    def dw(i, carry):
        pltpu.make_async_copy(rowt, out_hbm.at[pl.ds(0, TILE)], semrd).wait()
        return carry
    lax.fori_loop(0, 2 * w * nb * dt_n, dw, 0)


def kernel(tensor_list, row_embed_weight, col_embed_weight, depth_embed_weight):
    x = tensor_list
    h, w, d = x.shape[-3], x.shape[-2], x.shape[-1]
    nb = x.shape[0]
    f = row_embed_weight.shape[-1]
    n_chan = 3 * f

    info = plsc.get_sparse_core_info()
    nc, ns = info.num_cores, info.num_subcores
    assert nc * ns == h, "one vector subcore per h plane"

    colf = col_embed_weight.reshape(-1)
    rowf = row_embed_weight.reshape(-1)
    depf = depth_embed_weight.reshape(-1)

    dt_n = d // SUB
    ct_n = n_chan // LN
    total = nb * h * w * dt_n * ct_n * TILE

    run = pl.kernel(
        functools.partial(_pos_embed_body, nb, h, w, d, f, nc),
        mesh=plsc.VectorSubcoreMesh(core_axis_name="c", subcore_axis_name="s"),
        out_type=jax.ShapeDtypeStruct((total,), jnp.float32),
        scratch_types=[
            pltpu.VMEM((w * f,), jnp.float32),
            pltpu.VMEM((f,), jnp.float32),
            pltpu.VMEM((d * f,), jnp.float32),
            pltpu.VMEM((TILE,), jnp.float32),
            pltpu.VMEM((TILE,), jnp.float32),
            pltpu.VMEM((TILE,), jnp.float32),
            pltpu.SemaphoreType.DMA,
            pltpu.SemaphoreType.DMA,
            pltpu.SemaphoreType.DMA,
        ],
    )
    out = run(colf, rowf, depf)
    out7 = out.reshape(nb, h, w, dt_n, ct_n, SUB, LN)
    out5 = out7.transpose(0, 4, 6, 1, 2, 3, 5).reshape(nb, n_chan, h, w, d)
    return out5


# speedup vs baseline: 1.0982x; 1.0662x over previous
import functools

import jax
import jax.numpy as jnp
from jax import lax
from jax.experimental import pallas as pl
from jax.experimental.pallas import tpu as pltpu
from jax.experimental.pallas import tpu_sc as plsc

LANES = 16
SUB = 8
LN = 128
TILE = SUB * LN


def _pos_embed_body(nb, h, w, d, f, nc,
                    colf_hbm, rowf_hbm, depf_hbm, out_hbm,
                    colv, rowv, depv, rowt, colt0, colt1,
                    semc0, semc1, semrd):
    dt_n = d // SUB
    ct_n = (3 * f) // LN
    slab_len = dt_n * ct_n * TILE
    jn = f // LANES

    hh = lax.axis_index("s") * nc + lax.axis_index("c")

    pltpu.sync_copy(colf_hbm.at[pl.ds(0, w * f)], colv)
    pltpu.sync_copy(depf_hbm.at[pl.ds(0, d * f)], depv)
    pltpu.sync_copy(rowf_hbm.at[pl.ds(hh * f, f)], rowv)

    row_regs = [rowv[pl.ds(j * LANES, LANES)] for j in range(jn)]
    for dr in range(SUB):
        for j in range(jn):
            rowt[pl.ds(dr * LN + j * LANES, LANES)] = row_regs[j]

    def build_col(buf, wq):
        col_regs = [colv[pl.ds(wq * f + j * LANES, LANES)] for j in range(jn)]
        for dr in range(SUB):
            for j in range(jn):
                buf[pl.ds(dr * LN + j * LANES, LANES)] = col_regs[j]

    def fire(colt, wq, semc):
        for b in range(nb):
            base = ((b * h + hh) * w + wq) * slab_len
            for dt in range(dt_n):
                off = base + dt * ct_n * TILE
                pltpu.async_copy(colt, out_hbm.at[pl.ds(off, TILE)], semc)
                pltpu.async_copy(rowt, out_hbm.at[pl.ds(off + TILE, TILE)],
                                 semrd)
                pltpu.async_copy(depv.at[pl.ds(dt * TILE, TILE)],
                                 out_hbm.at[pl.ds(off + 2 * TILE, TILE)],
                                 semrd)

    def drain(buf, sem, n):
        for _ in range(n):
            pltpu.make_async_copy(buf, out_hbm.at[pl.ds(0, TILE)], sem).wait()

    col_fires = nb * dt_n

    build_col(colt0, 0)
    fire(colt0, 0, semc0)
    build_col(colt1, 1)
    fire(colt1, 1, semc1)

    def pair(k, carry):
        wq = 2 * k
        drain(colt0, semc0, col_fires)
        build_col(colt0, wq)
        fire(colt0, wq, semc0)
        drain(colt1, semc1, col_fires)
        build_col(colt1, wq + 1)
        fire(colt1, wq + 1, semc1)
        return carry

    lax.fori_loop(1, w // 2, pair, 0)

    drain(colt0, semc0, col_fires)
    drain(colt1, semc1, col_fires)

    def dw(i, carry):
        pltpu.make_async_copy(rowt, out_hbm.at[pl.ds(0, TILE)], semrd).wait()
        return carry
    lax.fori_loop(0, 2 * w * nb * dt_n, dw, 0)


def kernel(tensor_list, row_embed_weight, col_embed_weight, depth_embed_weight):
    x = tensor_list
    h, w, d = x.shape[-3], x.shape[-2], x.shape[-1]
    nb = x.shape[0]
    f = row_embed_weight.shape[-1]
    n_chan = 3 * f

    info = plsc.get_sparse_core_info()
    nc, ns = info.num_cores, info.num_subcores
    assert nc * ns == h, "one vector subcore per h plane"

    colf = col_embed_weight.reshape(-1)
    rowf = row_embed_weight.reshape(-1)
    depf = depth_embed_weight.reshape(-1)

    dt_n = d // SUB
    ct_n = n_chan // LN
    total = nb * h * w * dt_n * ct_n * TILE

    run = pl.kernel(
        functools.partial(_pos_embed_body, nb, h, w, d, f, nc),
        mesh=plsc.VectorSubcoreMesh(core_axis_name="c", subcore_axis_name="s"),
        out_type=jax.ShapeDtypeStruct((total,), jnp.float32),
        scratch_types=[
            pltpu.VMEM((w * f,), jnp.float32),
            pltpu.VMEM((f,), jnp.float32),
            pltpu.VMEM((d * f,), jnp.float32),
            pltpu.VMEM((TILE,), jnp.float32),
            pltpu.VMEM((TILE,), jnp.float32),
            pltpu.VMEM((TILE,), jnp.float32),
            pltpu.SemaphoreType.DMA,
            pltpu.SemaphoreType.DMA,
            pltpu.SemaphoreType.DMA,
        ],
    )
    out = run(colf, rowf, depf)
    out7 = out.reshape(nb, h, w, dt_n, ct_n, SUB, LN)
    out5 = out7.transpose(0, 4, 6, 1, 2, 3, 5).reshape(nb, n_chan, h, w, d)
    return out5


# speedup vs baseline: 1.1272x; 1.0264x over previous
import functools

import jax
import jax.numpy as jnp
from jax import lax
from jax.experimental import pallas as pl
from jax.experimental.pallas import tpu as pltpu
from jax.experimental.pallas import tpu_sc as plsc

LANES = 16
SUB = 8
LN = 128
TILE = SUB * LN


def _pos_embed_body(nb, h, w, d, f, nc,
                    colf_hbm, rowf_hbm, depf_hbm, out_hbm,
                    colv, rowv, depv, rowt, colt0, colt1,
                    semc0, semc1, semrd):
    dt_n = d // SUB
    ct_n = (3 * f) // LN
    slab_len = dt_n * ct_n * TILE
    jn = f // LANES

    hh = lax.axis_index("s") * nc + lax.axis_index("c")

    pltpu.sync_copy(colf_hbm.at[pl.ds(0, w * f)], colv)
    pltpu.sync_copy(depf_hbm.at[pl.ds(0, d * f)], depv)
    pltpu.sync_copy(rowf_hbm.at[pl.ds(hh * f, f)], rowv)

    row_regs = [rowv[pl.ds(j * LANES, LANES)] for j in range(jn)]
    for dr in range(SUB):
        for j in range(jn):
            rowt[pl.ds(dr * LN + j * LANES, LANES)] = row_regs[j]

    def build_col(buf, wq):
        col_regs = [colv[pl.ds(wq * f + j * LANES, LANES)] for j in range(jn)]
        for dr in range(SUB):
            for j in range(jn):
                buf[pl.ds(dr * LN + j * LANES, LANES)] = col_regs[j]

    def fire(colt, wq, semc):
        for b in range(nb):
            base = ((b * h + hh) * w + wq) * slab_len
            for dt in range(dt_n):
                off = base + dt * ct_n * TILE
                pltpu.async_copy(colt, out_hbm.at[pl.ds(off, TILE)], semc)
                pltpu.async_copy(rowt, out_hbm.at[pl.ds(off + TILE, TILE)],
                                 semrd)
                pltpu.async_copy(depv.at[pl.ds(dt * TILE, TILE)],
                                 out_hbm.at[pl.ds(off + 2 * TILE, TILE)],
                                 semrd)

    def drain(n, sem):
        for _ in range(n // 4):
            pltpu.make_async_copy(
                colv, out_hbm.at[pl.ds(0, 4 * TILE)], sem).wait()
        for _ in range(n % 4):
            pltpu.make_async_copy(
                rowt, out_hbm.at[pl.ds(0, TILE)], sem).wait()

    col_fires = nb * dt_n

    build_col(colt0, 0)
    fire(colt0, 0, semc0)
    build_col(colt1, 1)
    fire(colt1, 1, semc1)

    def pair(k, carry):
        wq = 2 * k
        drain(col_fires, semc0)
        build_col(colt0, wq)
        fire(colt0, wq, semc0)
        drain(col_fires, semc1)
        build_col(colt1, wq + 1)
        fire(colt1, wq + 1, semc1)
        return carry

    lax.fori_loop(1, w // 2, pair, 0)

    drain(col_fires, semc0)
    drain(col_fires, semc1)

    def dw(i, carry):
        pltpu.make_async_copy(
            colv, out_hbm.at[pl.ds(0, 4 * TILE)], semrd).wait()
        return carry
    lax.fori_loop(0, 2 * w * nb * dt_n // 4, dw, 0)


def kernel(tensor_list, row_embed_weight, col_embed_weight, depth_embed_weight):
    x = tensor_list
    h, w, d = x.shape[-3], x.shape[-2], x.shape[-1]
    nb = x.shape[0]
    f = row_embed_weight.shape[-1]
    n_chan = 3 * f

    info = plsc.get_sparse_core_info()
    nc, ns = info.num_cores, info.num_subcores
    assert nc * ns == h, "one vector subcore per h plane"

    colf = col_embed_weight.reshape(-1)
    rowf = row_embed_weight.reshape(-1)
    depf = depth_embed_weight.reshape(-1)

    dt_n = d // SUB
    ct_n = n_chan // LN
    total = nb * h * w * dt_n * ct_n * TILE

    run = pl.kernel(
        functools.partial(_pos_embed_body, nb, h, w, d, f, nc),
        mesh=plsc.VectorSubcoreMesh(core_axis_name="c", subcore_axis_name="s"),
        out_type=jax.ShapeDtypeStruct((total,), jnp.float32),
        scratch_types=[
            pltpu.VMEM((w * f,), jnp.float32),
            pltpu.VMEM((f,), jnp.float32),
            pltpu.VMEM((d * f,), jnp.float32),
            pltpu.VMEM((TILE,), jnp.float32),
            pltpu.VMEM((TILE,), jnp.float32),
            pltpu.VMEM((TILE,), jnp.float32),
            pltpu.SemaphoreType.DMA,
            pltpu.SemaphoreType.DMA,
            pltpu.SemaphoreType.DMA,
        ],
    )
    out = run(colf, rowf, depf)
    out7 = out.reshape(nb, h, w, dt_n, ct_n, SUB, LN)
    out5 = out7.transpose(0, 4, 6, 1, 2, 3, 5).reshape(nb, n_chan, h, w, d)
    return out5
